# baseline (speedup 1.0000x reference)
import functools
import math

import jax
import jax.numpy as jnp
from jax import lax
from jax.experimental import pallas as pl
from jax.experimental.pallas import tpu as pltpu
from jax.experimental.pallas import tpu_sc as plsc

D = 32
SCALE = math.sqrt(float(D))
CHUNK = 128
NC, NS, L = 2, 16, 16
NW = NC * NS


@functools.lru_cache(maxsize=None)
def _build(B):
    nrow = B // CHUNK
    rpw = nrow // NW

    mesh = plsc.VectorSubcoreMesh(core_axis_name="c", subcore_axis_name="s")

    @functools.partial(
        pl.kernel,
        mesh=mesh,
        compiler_params=pltpu.CompilerParams(use_tc_tiling_on_sc=False),
        out_type=jax.ShapeDtypeStruct((B, D), jnp.float32),
        scratch_types=[
            pltpu.VMEM((rpw, CHUNK), jnp.int32),
            pltpu.VMEM((CHUNK, D), jnp.float32),
            pltpu.VMEM((CHUNK, D), jnp.float32),
            pltpu.SemaphoreType.DMA,
            pltpu.SemaphoreType.DMA,
        ],
    )
    def emb(x_hbm, ids_hbm, tab_hbm, out_hbm, idx_v, g_v, x_v, gsem, xsem):
        wid = lax.axis_index("s") * NC + lax.axis_index("c")
        r0 = wid * rpw
        pltpu.sync_copy(ids_hbm.at[pl.ds(r0, rpw)], idx_v)

        def chunk_body(j, carry):
            base = (r0 + j) * CHUNK
            gcp = pltpu.async_copy(tab_hbm.at[idx_v.at[j]], g_v, gsem)
            xcp = pltpu.async_copy(x_hbm.at[pl.ds(base, CHUNK)], x_v, xsem)
            gcp.wait()
            xcp.wait()

            def row_body(r, c2):
                for h in range(D // L):
                    sl = (r, pl.ds(h * L, L))
                    x_v[sl] = x_v[sl] + g_v[sl] * SCALE
                return c2

            lax.fori_loop(0, CHUNK, row_body, 0)
            pltpu.sync_copy(x_v, out_hbm.at[pl.ds(base, CHUNK)])
            return carry

        lax.fori_loop(0, rpw, chunk_body, 0)

    return emb


def kernel(x, ids, table):
    B = x.shape[0] * x.shape[1]
    xf = x.reshape(B, D)
    idsf = ids.reshape(B // CHUNK, CHUNK).astype(jnp.int32)
    out = _build(B)(xf, idsf, table)
    return out.reshape(x.shape)


# speedup vs baseline: 1.1957x; 1.1957x over previous
import functools
import math

import jax
import jax.numpy as jnp
from jax import lax
from jax.experimental import pallas as pl
from jax.experimental.pallas import tpu as pltpu
from jax.experimental.pallas import tpu_sc as plsc

D = 32
SCALE = math.sqrt(float(D))
CHUNK = 128
K = 4
NC, NS, L = 2, 16, 16
NW = NC * NS


@functools.lru_cache(maxsize=None)
def _build(B):
    nrow = B // CHUNK
    rpw = nrow // NW
    ng = rpw // K

    mesh = plsc.VectorSubcoreMesh(core_axis_name="c", subcore_axis_name="s")

    @functools.partial(
        pl.kernel,
        mesh=mesh,
        compiler_params=pltpu.CompilerParams(use_tc_tiling_on_sc=False),
        out_type=jax.ShapeDtypeStruct((B, D), jnp.float32),
        scratch_types=[
            pltpu.VMEM((rpw, CHUNK), jnp.int32),
            pltpu.VMEM((2, K, CHUNK, D), jnp.float32),
            pltpu.VMEM((2, K, CHUNK, D), jnp.float32),
            pltpu.VMEM((2, K, CHUNK, D), jnp.float32),
            pltpu.SemaphoreType.DMA,
            pltpu.SemaphoreType.DMA,
            pltpu.SemaphoreType.DMA,
        ],
    )
    def emb(x_hbm, ids_hbm, tab_hbm, out_hbm, idx_v, g_v, x_v, o_v,
            gsem, xsem, osem):
        wid = lax.axis_index("s") * NC + lax.axis_index("c")
        r0 = wid * rpw
        pltpu.sync_copy(ids_hbm.at[pl.ds(r0, rpw)], idx_v)

        def issue_in(g, p):
            for k in range(K):
                j = g * K + k
                base = (r0 + j) * CHUNK
                pltpu.async_copy(tab_hbm.at[idx_v.at[j]], g_v.at[p, k], gsem)
                pltpu.async_copy(x_hbm.at[pl.ds(base, CHUNK)], x_v.at[p, k],
                                 xsem)

        def drain_in(p):
            for k in range(K):
                pltpu.make_async_copy(x_hbm.at[pl.ds(0, CHUNK)],
                                      g_v.at[p, k], gsem).wait()
                pltpu.make_async_copy(x_hbm.at[pl.ds(0, CHUNK)],
                                      x_v.at[p, k], xsem).wait()

        def issue_out(g, p):
            for k in range(K):
                base = (r0 + g * K + k) * CHUNK
                pltpu.async_copy(o_v.at[p, k], out_hbm.at[pl.ds(base, CHUNK)],
                                 osem)

        def drain_out(p):
            for k in range(K):
                pltpu.make_async_copy(o_v.at[p, k],
                                      out_hbm.at[pl.ds(0, CHUNK)], osem).wait()

        def group_body(g, p):
            @pl.when(g + 1 < ng)
            def _():
                issue_in(g + 1, 1 - p)

            drain_in(p)

            @pl.when(g >= 2)
            def _():
                drain_out(p)

            @plsc.parallel_loop(0, CHUNK, unroll=4)
            def _(r):
                for k in range(K):
                    for h in range(D // L):
                        sl = (p, k, r, pl.ds(h * L, L))
                        o_v[sl] = x_v[sl] + g_v[sl] * SCALE

            issue_out(g, p)

        issue_in(0, 0)

        def outer(g2, carry):
            for p in range(2):
                group_body(g2 * 2 + p, p)
            return carry

        lax.fori_loop(0, ng // 2, outer, 0)
        drain_out(0)
        drain_out(1)

    return emb


def kernel(x, ids, table):
    B = x.shape[0] * x.shape[1]
    xf = x.reshape(B, D)
    idsf = ids.reshape(B // CHUNK, CHUNK).astype(jnp.int32)
    out = _build(B)(xf, idsf, table)
    return out.reshape(x.shape)


# speedup vs baseline: 1.5663x; 1.3100x over previous
import functools
import math

import jax
import jax.numpy as jnp
from jax import lax
from jax.experimental import pallas as pl
from jax.experimental.pallas import tpu as pltpu
from jax.experimental.pallas import tpu_sc as plsc

D = 32
SCALE = math.sqrt(float(D))
CHUNK = 128
K = 4
NC, NS, L = 2, 16, 16
NW = NC * NS
CHI, CLO = 4, 8
SLAB = CLO * CHUNK


@functools.lru_cache(maxsize=None)
def _build(T, BH):
    assert BH == NW
    ng = T // K

    mesh = plsc.VectorSubcoreMesh(core_axis_name="c", subcore_axis_name="s")

    @functools.partial(
        pl.kernel,
        mesh=mesh,
        compiler_params=pltpu.CompilerParams(use_tc_tiling_on_sc=False,
                                             needs_layout_passes=False),
        out_type=jax.ShapeDtypeStruct((T, CHI, BH, SLAB), jnp.float32),
        scratch_types=[
            pltpu.VMEM((T, CHUNK), jnp.int32),
            pltpu.VMEM((2, K, CHUNK, D), jnp.float32),
            pltpu.VMEM((2, K, CHI, SLAB), jnp.float32),
            pltpu.VMEM((2, K, CHI, SLAB), jnp.float32),
            pltpu.SemaphoreType.DMA,
            pltpu.SemaphoreType.DMA,
            pltpu.SemaphoreType.DMA,
        ],
    )
    def emb(xs, idt, tab, outs, idx_v, g_v, x_v, o_v, gsem, xsem, osem):
        w = lax.axis_index("s") * NC + lax.axis_index("c")
        pltpu.sync_copy(idt.at[:, pl.ds(w * CHUNK, CHUNK)], idx_v)

        def issue_in(g, p):
            for k in range(K):
                t = g * K + k
                pltpu.async_copy(tab.at[idx_v.at[t]], g_v.at[p, k], gsem)
                pltpu.async_copy(xs.at[t, :, w], x_v.at[p, k], xsem)

        def drain_in(g, p):
            for k in range(K):
                pltpu.make_async_copy(tab.at[idx_v.at[g * K + k]],
                                      g_v.at[p, k], gsem).wait()
                pltpu.make_async_copy(xs.at[0, :, 0], x_v.at[p, k],
                                      xsem).wait()

        def issue_out(g, p):
            for k in range(K):
                t = g * K + k
                pltpu.async_copy(o_v.at[p, k], outs.at[t, :, w], osem)

        def drain_out(p):
            for k in range(K):
                pltpu.make_async_copy(o_v.at[p, k], outs.at[0, :, 0],
                                      osem).wait()

        lanes = lax.iota(jnp.int32, L)

        def group_body(g, p):
            @pl.when(g + 1 < ng)
            def _():
                issue_in(g + 1, 1 - p)

            drain_in(g, p)

            @pl.when(g >= 2)
            def _():
                drain_out(p)

            for k in range(K):

                @plsc.parallel_loop(0, CHUNK // L, unroll=1)
                def _(i):
                    rows = i * L + lanes
                    zero = rows * 0
                    for chi in range(CHI):
                        for clo in range(CLO):
                            c = chi * CLO + clo
                            vals = plsc.load_gather(g_v.at[p, k],
                                                    [rows, zero + c])
                            sl = (p, k, chi, pl.ds(clo * CHUNK + i * L, L))
                            o_v[sl] = x_v[sl] + vals * SCALE

            issue_out(g, p)

        issue_in(0, 0)

        def outer(g2, carry):
            for p in range(2):
                group_body(g2 * 2 + p, p)
            return carry

        lax.fori_loop(0, ng // 2, outer, 0)
        drain_out(0)
        drain_out(1)

    return emb


def kernel(x, ids, table):
    B0, T = x.shape[0], x.shape[1]
    bh = B0 // CHUNK
    xs = (x.reshape(bh, CHUNK, T, CHI, CLO).transpose(2, 3, 0, 4, 1)
          .reshape(T, CHI, bh, SLAB))
    idt = ids.transpose(1, 0).astype(jnp.int32)
    res = _build(T, bh)(xs, idt, table)
    return (res.reshape(T, CHI, bh, CLO, CHUNK).transpose(2, 4, 0, 1, 3)
            .reshape(B0, T, D))


# speedup vs baseline: 2.3730x; 1.5150x over previous
import functools
import math

import jax
import jax.numpy as jnp
from jax import lax
from jax.experimental import pallas as pl
from jax.experimental.pallas import tpu as pltpu
from jax.experimental.pallas import tpu_sc as plsc

D = 32
SCALE = math.sqrt(float(D))
CHUNK = 128
K = 4
NC, NS, L = 2, 16, 16
NW = NC * NS
CHI, CLO = 4, 8
SLAB = CLO * CHUNK


@functools.lru_cache(maxsize=None)
def _build(T, BH):
    assert BH == NW
    ng = T // K

    mesh = plsc.VectorSubcoreMesh(core_axis_name="c", subcore_axis_name="s")

    @functools.partial(
        pl.kernel,
        mesh=mesh,
        compiler_params=pltpu.CompilerParams(use_tc_tiling_on_sc=False,
                                             needs_layout_passes=False),
        out_type=jax.ShapeDtypeStruct((T, CHI, BH, SLAB), jnp.float32),
        scratch_types=[
            pltpu.VMEM((T, CHUNK), jnp.int32),
            pltpu.VMEM((2, K, CHUNK, D), jnp.float32),
            pltpu.VMEM((2, K, CHI, SLAB), jnp.float32),
            pltpu.VMEM((2, K, CHI, SLAB), jnp.float32),
            pltpu.SemaphoreType.DMA,
            pltpu.SemaphoreType.DMA,
            pltpu.SemaphoreType.DMA,
        ],
    )
    def emb(xs, idt, tab, outs, idx_v, g_v, x_v, o_v, gsem, xsem, osem):
        w = lax.axis_index("s") * NC + lax.axis_index("c")
        pltpu.sync_copy(idt.at[:, pl.ds(w * CHUNK, CHUNK)], idx_v)

        def issue_in(g, p):
            for k in range(K):
                t = g * K + k
                pltpu.async_copy(tab.at[idx_v.at[t]], g_v.at[p, k], gsem)
                pltpu.async_copy(xs.at[t, :, w], x_v.at[p, k], xsem)

        def drain_in(g, p):
            for k in range(K):
                pltpu.make_async_copy(tab.at[idx_v.at[g * K + k]],
                                      g_v.at[p, k], gsem).wait()
                pltpu.make_async_copy(xs.at[0, :, 0], x_v.at[p, k],
                                      xsem).wait()

        def issue_out(g, p):
            for k in range(K):
                t = g * K + k
                pltpu.async_copy(o_v.at[p, k], outs.at[t, :, w], osem)

        def drain_out(p):
            for k in range(K):
                pltpu.make_async_copy(o_v.at[p, k], outs.at[0, :, 0],
                                      osem).wait()

        lanes = lax.iota(jnp.int32, L)

        def group_body(g, p):
            @pl.when(g + 1 < ng)
            def _():
                issue_in(g + 1, 1 - p)

            drain_in(g, p)

            @pl.when(g >= 2)
            def _():
                drain_out(p)

            for k in range(K):
                gk = g_v.at[p, k]
                xk = x_v.at[p, k]
                ok = o_v.at[p, k]

                @plsc.parallel_loop(0, D, unroll=1)
                def _(c0):
                    c_vec = (c0 + lanes) & (D - 1)
                    chi_vec = c_vec >> 3
                    inner_base = (c_vec & (CLO - 1)) * CHUNK
                    for i in range(CHUNK // L):
                        rows = i * L + lanes
                        inner = inner_base + rows
                        vals = plsc.load_gather(gk, [rows, c_vec])
                        xv = plsc.load_gather(xk, [chi_vec, inner])
                        plsc.store_scatter(ok, [chi_vec, inner],
                                           xv + vals * SCALE)

            issue_out(g, p)

        issue_in(0, 0)

        def outer(g2, carry):
            for p in range(2):
                group_body(g2 * 2 + p, p)
            return carry

        lax.fori_loop(0, ng // 2, outer, 0)
        drain_out(0)
        drain_out(1)

    return emb


def kernel(x, ids, table):
    B0, T = x.shape[0], x.shape[1]
    bh = B0 // CHUNK
    xs = (x.reshape(bh, CHUNK, T, CHI, CLO).transpose(2, 3, 0, 4, 1)
          .reshape(T, CHI, bh, SLAB))
    idt = ids.transpose(1, 0).astype(jnp.int32)
    res = _build(T, bh)(xs, idt, table)
    return (res.reshape(T, CHI, bh, CLO, CHUNK).transpose(2, 4, 0, 1, 3)
            .reshape(B0, T, D))
